# Initial kernel scaffold; baseline (speedup 1.0000x reference)
import jax
import jax.numpy as jnp
from jax.experimental import pallas as pl


def kernel(x, edge_index, edge_attr, params):
    raise NotImplementedError("write your pallas kernel here")


# speedup vs baseline: 10.5691x; 10.5691x over previous
import functools
import math

import jax
import jax.numpy as jnp
from jax import lax
from jax.experimental import pallas as pl
from jax.experimental.pallas import tpu as pltpu
from jax.experimental.pallas import tpu_sc as plsc

N = 10000
E = 320000
D = 128
VW = 144
NC = 2
NS = 16
NW = NC * NS
EPT = E // NW
EPC = E // NC
NPT = N // NS
BLK1 = 2000
C2 = 80
TB = 1024
NP = 10240
GRID = (N + TB - 1) // TB

_f32 = jnp.float32
_i32 = jnp.int32
_HI = jax.lax.Precision.HIGHEST


def _mm(a, b, dims):
    return lax.dot_general(a, b, (dims, ((), ())), precision=_HI,
                           preferred_element_type=_f32)


def _prep_body(h_ref, wq, bq, wk, bk, wv, bv, ws, bs,
               qkT_ref, vext_ref, sx_ref):
    hb = h_ref[...]
    _emit_prep(hb, wq, bq, wk, bk, wv, bv, ws, bs, qkT_ref, vext_ref, sx_ref)


def _emit_prep(hb, wq, bq, wk, bk, wv, bv, ws, bs, qkT_ref, vext_ref, sx_ref):
    inv = 1.0 / math.sqrt(D)
    a8 = _mm(wq[...], wk[...], ((1,), (1,)))
    a1 = _mm(wq[...], bk[...], ((1,), (1,)))
    a9 = jnp.concatenate([a8, a1], axis=1) * inv
    b8 = _mm(bq[...], wk[...], ((1,), (1,)))
    b1 = _mm(bq[...], bk[...], ((1,), (1,)))
    b9 = jnp.concatenate([b8, b1], axis=1) * inv
    qkT_ref[...] = _mm(a9, hb, ((0,), (1,))) + b9.reshape(9, 1)
    v = _mm(hb, wv[...], ((1,), (0,))) + bv[...]
    ones = jnp.ones((hb.shape[0], 1), _f32)
    zer = jnp.zeros((hb.shape[0], VW - D - 1), _f32)
    vext_ref[...] = jnp.concatenate([v, ones, zer], axis=1)
    sx_ref[...] = _mm(hb, ws[...], ((1,), (0,))) + bs[...]


def _merge_prep_body(agg_ref, sxp_ref, wq, bq, wk, bk, wv, bv, ws, bs,
                     qkT_ref, vext_ref, sx_ref):
    p = agg_ref[0] + agg_ref[1]
    h = p[:, :D] / (p[:, D:D + 1] + 1e-16) + sxp_ref[...]
    h = jnp.maximum(h, 0.0)
    _emit_prep(h, wq, bq, wk, bk, wv, bv, ws, bs, qkT_ref, vext_ref, sx_ref)


def _final_body(agg_ref, sxp_ref, out_ref):
    p = agg_ref[0] + agg_ref[1]
    out_ref[...] = p[:, :D] / (p[:, D:D + 1] + 1e-16) + sxp_ref[...]


def _smax_reduce_body(tiles_ref, out_ref):
    g = jnp.max(tiles_ref[...], axis=0)
    out_ref[...] = jnp.where(jnp.isfinite(g), g, 0.0)


def _w_specs():
    shapes = [(D, D), (1, D), (8, D), (1, D), (D, D), (1, D), (D, D), (1, D)]
    return [pl.BlockSpec(s, lambda i: (0, 0)) for s in shapes]


def _tc_prep(h, w):
    return pl.pallas_call(
        _prep_body,
        grid=(GRID,),
        in_specs=[pl.BlockSpec((TB, D), lambda i: (i, 0))] + _w_specs(),
        out_specs=[
            pl.BlockSpec((9, TB), lambda i: (0, i)),
            pl.BlockSpec((TB, VW), lambda i: (i, 0)),
            pl.BlockSpec((TB, D), lambda i: (i, 0)),
        ],
        out_shape=[
            jax.ShapeDtypeStruct((9, NP), _f32),
            jax.ShapeDtypeStruct((N, VW), _f32),
            jax.ShapeDtypeStruct((N, D), _f32),
        ],
    )(h, *w)


def _tc_merge_prep(agg, sxp, w):
    return pl.pallas_call(
        _merge_prep_body,
        grid=(GRID,),
        in_specs=[pl.BlockSpec((NC, TB, VW), lambda i: (0, i, 0)),
                  pl.BlockSpec((TB, D), lambda i: (i, 0))] + _w_specs(),
        out_specs=[
            pl.BlockSpec((9, TB), lambda i: (0, i)),
            pl.BlockSpec((TB, VW), lambda i: (i, 0)),
            pl.BlockSpec((TB, D), lambda i: (i, 0)),
        ],
        out_shape=[
            jax.ShapeDtypeStruct((9, NP), _f32),
            jax.ShapeDtypeStruct((N, VW), _f32),
            jax.ShapeDtypeStruct((N, D), _f32),
        ],
    )(agg, sxp, *w)


def _tc_final(agg, sxp):
    return pl.pallas_call(
        _final_body,
        grid=(GRID,),
        in_specs=[pl.BlockSpec((NC, TB, VW), lambda i: (0, i, 0)),
                  pl.BlockSpec((TB, D), lambda i: (i, 0))],
        out_specs=pl.BlockSpec((TB, D), lambda i: (i, 0)),
        out_shape=jax.ShapeDtypeStruct((N, D), _f32),
    )(agg, sxp)


def _tc_smax_reduce(tiles):
    return pl.pallas_call(
        _smax_reduce_body,
        grid=(1,),
        in_specs=[pl.BlockSpec((NW, N), lambda i: (0, 0))],
        out_specs=pl.BlockSpec((N,), lambda i: (0,)),
        out_shape=jax.ShapeDtypeStruct((N,), _f32),
    )(tiles)


def _sc_pass1_body(qkT, eaT, dst, s_out, smax_tiles,
                   planes_v, ea_v, dst_v, s_v, smax_v):
    c = lax.axis_index("c")
    t = lax.axis_index("s")
    wid = c * NS + t
    base_e = c * EPC + t * EPT

    for d in range(9):
        pltpu.sync_copy(qkT.at[pl.ds(d * NP, N)], planes_v.at[d])

    def _init(i, carry):
        smax_v[pl.ds(i * 16, 16)] = jnp.full((16,), -jnp.inf, _f32)
        return carry
    lax.fori_loop(0, N // 16, _init, 0)

    for blk in range(EPT // BLK1):
        b0 = base_e + blk * BLK1
        pltpu.sync_copy(dst.at[pl.ds(b0, BLK1)], dst_v)
        for d in range(8):
            pltpu.sync_copy(eaT.at[pl.ds(d * E + b0, BLK1)], ea_v.at[d])

        def _grp(j, carry):
            dstv = dst_v[pl.ds(j * 16, 16)]
            sacc = plsc.load_gather(planes_v, [jnp.full((16,), 8, _i32), dstv])
            for d in range(8):
                qd = plsc.load_gather(
                    planes_v, [jnp.full((16,), d, _i32), dstv])
                sacc = sacc + qd * ea_v[d, pl.ds(j * 16, 16)]
            s_v[pl.ds(j * 16, 16)] = sacc
            cur = plsc.load_gather(smax_v, [dstv])
            pend = sacc > cur

            def _cond(p):
                return jnp.any(p)

            def _body(p):
                plsc.store_scatter(smax_v, [dstv], sacc, mask=p)
                cur2 = plsc.load_gather(smax_v, [dstv])
                return p & (sacc > cur2)

            lax.while_loop(_cond, _body, pend)
            return carry
        lax.fori_loop(0, BLK1 // 16, _grp, 0)
        pltpu.sync_copy(s_v, s_out.at[pl.ds(b0, BLK1)])

    pltpu.sync_copy(smax_v, smax_tiles.at[pl.ds(wid * N, N)])


def _sc_pass1(qkT, eaT, dst):
    mesh = plsc.VectorSubcoreMesh(core_axis_name="c", subcore_axis_name="s")
    f = pl.kernel(
        _sc_pass1_body,
        out_type=[
            jax.ShapeDtypeStruct((E,), _f32),
            jax.ShapeDtypeStruct((NW * N,), _f32),
        ],
        mesh=mesh,
        scratch_types=[
            pltpu.VMEM((9, N), _f32),
            pltpu.VMEM((8, BLK1), _f32),
            pltpu.VMEM((BLK1,), _i32),
            pltpu.VMEM((BLK1,), _f32),
            pltpu.VMEM((N,), _f32),
        ],
        compiler_params=pltpu.CompilerParams(use_tc_tiling_on_sc=False, needs_layout_passes=False),
    )
    return f(qkT, eaT, dst)


def _sc_pass2_body(src, dst, s_all, gsmax, vext, agg,
                   smax_v, rows, sidx, didx, sbuf, exbuf, zbuf, sem, agg_s):
    c = lax.axis_index("c")
    t = lax.axis_index("s")
    base_e = c * EPC + t * EPT

    pltpu.sync_copy(gsmax, smax_v)

    for r in range(25):
        for g in range(VW // 16):
            zbuf[r, pl.ds(g * 16, 16)] = jnp.zeros((16,), _f32)

    def _z(k, carry):
        pltpu.sync_copy(zbuf, agg_s.at[pl.ds(t * NPT + k * 25, 25)])
        return carry
    lax.fori_loop(0, NPT // 25, _z, 0)
    plsc.subcore_barrier()

    def _chunk(i, carry):
        cb = base_e + i * C2
        pltpu.sync_copy(src.at[pl.ds(cb, C2)], sidx)
        pltpu.sync_copy(dst.at[pl.ds(cb, C2)], didx)
        pltpu.sync_copy(s_all.at[pl.ds(cb, C2)], sbuf)
        pltpu.async_copy(vext.at[sidx], rows, sem).wait()
        for b in range(C2 // 16):
            dstv = didx[pl.ds(b * 16, 16)]
            sm = plsc.load_gather(smax_v, [dstv])
            exbuf[pl.ds(b * 16, 16)] = jnp.exp(sbuf[pl.ds(b * 16, 16)] - sm)
        lane0 = lax.broadcasted_iota(_i32, (16,), 0) == 0
        for e in range(C2):
            exb = plsc.load_gather(exbuf, [jnp.full((16,), e, _i32)])
            for g in range(D // 16):
                rows[e, pl.ds(g * 16, 16)] = rows[e, pl.ds(g * 16, 16)] * exb
            rows[e, pl.ds(D, 16)] = jnp.where(lane0, exb, 0.0)
        pltpu.sync_copy(rows, agg_s.at[didx], add=True)
        return carry
    lax.fori_loop(0, EPT // C2, _chunk, 0)
    plsc.subcore_barrier()

    pltpu.sync_copy(agg_s.at[pl.ds(t * NPT, NPT)],
                    agg.at[c, pl.ds(t * NPT, NPT)])


def _sc_pass2(src, dst, s_all, gsmax, vext):
    mesh = plsc.VectorSubcoreMesh(core_axis_name="c", subcore_axis_name="s")
    f = pl.kernel(
        _sc_pass2_body,
        out_type=jax.ShapeDtypeStruct((NC, N, VW), _f32),
        mesh=mesh,
        scratch_types=[
            pltpu.VMEM((N,), _f32),
            pltpu.VMEM((C2, VW), _f32),
            pltpu.VMEM((C2,), _i32),
            pltpu.VMEM((C2,), _i32),
            pltpu.VMEM((C2,), _f32),
            pltpu.VMEM((C2,), _f32),
            pltpu.VMEM((25, VW), _f32),
            pltpu.SemaphoreType.DMA,
            pltpu.VMEM_SHARED((N, VW), _f32),
        ],
        compiler_params=pltpu.CompilerParams(use_tc_tiling_on_sc=False, needs_layout_passes=False),
    )
    return f(src, dst, s_all, gsmax, vext)


def kernel(x, edge_index, edge_attr, params):
    src = edge_index[0]
    dst = edge_index[1]
    eaT = edge_attr.T.reshape(-1)

    def weights(p):
        return (p["Wq"], p["bq"].reshape(1, D), p["Wk"], p["bk"].reshape(1, D),
                p["Wv"], p["bv"].reshape(1, D), p["Ws"], p["bs"].reshape(1, D))

    agg = None
    sx = None
    for li in range(3):
        w = weights(params[li])
        if li == 0:
            qkT, vext, sx = _tc_prep(x, w)
        else:
            qkT, vext, sx = _tc_merge_prep(agg, sx, w)
        s_all, smax_tiles = _sc_pass1(qkT.reshape(-1), eaT, dst)
        gsmax = _tc_smax_reduce(smax_tiles.reshape(NW, N))
        agg = _sc_pass2(src, dst, s_all, gsmax, vext)
    return _tc_final(agg, sx)


# speedup vs baseline: 12.9154x; 1.2220x over previous
import functools
import math

import jax
import jax.numpy as jnp
from jax import lax
from jax.experimental import pallas as pl
from jax.experimental.pallas import tpu as pltpu
from jax.experimental.pallas import tpu_sc as plsc

N = 10000
E = 320000
D = 128
VW = 144
NC = 2
NS = 16
NW = NC * NS
EPT = E // NW
EPC = E // NC
NPT = N // NS
BLK1 = 2000
C2 = 80
TB = 1024
NP = 10240
GRID = (N + TB - 1) // TB

_f32 = jnp.float32
_i32 = jnp.int32
_HI = jax.lax.Precision.HIGHEST


def _mm(a, b, dims):
    return lax.dot_general(a, b, (dims, ((), ())), precision=_HI,
                           preferred_element_type=_f32)


def _prep_body(h_ref, wq, bq, wk, bk, wv, bv, ws, bs,
               qkT_ref, vext_ref, sx_ref):
    hb = h_ref[...]
    _emit_prep(hb, wq, bq, wk, bk, wv, bv, ws, bs, qkT_ref, vext_ref, sx_ref)


def _emit_prep(hb, wq, bq, wk, bk, wv, bv, ws, bs, qkT_ref, vext_ref, sx_ref):
    inv = 1.0 / math.sqrt(D)
    a8 = _mm(wq[...], wk[...], ((1,), (1,)))
    a1 = _mm(wq[...], bk[...], ((1,), (1,)))
    a9 = jnp.concatenate([a8, a1], axis=1) * inv
    b8 = _mm(bq[...], wk[...], ((1,), (1,)))
    b1 = _mm(bq[...], bk[...], ((1,), (1,)))
    b9 = jnp.concatenate([b8, b1], axis=1) * inv
    qkT_ref[...] = _mm(a9, hb, ((0,), (1,))) + b9.reshape(9, 1)
    v = _mm(hb, wv[...], ((1,), (0,))) + bv[...]
    ones = jnp.ones((hb.shape[0], 1), _f32)
    zer = jnp.zeros((hb.shape[0], VW - D - 1), _f32)
    vext_ref[...] = jnp.concatenate([v, ones, zer], axis=1)
    sx_ref[...] = _mm(hb, ws[...], ((1,), (0,))) + bs[...]


def _merge_prep_body(agg_ref, sxp_ref, wq, bq, wk, bk, wv, bv, ws, bs,
                     qkT_ref, vext_ref, sx_ref):
    p = agg_ref[0] + agg_ref[1]
    h = p[:, :D] / (p[:, D:D + 1] + 1e-16) + sxp_ref[...]
    h = jnp.maximum(h, 0.0)
    _emit_prep(h, wq, bq, wk, bk, wv, bv, ws, bs, qkT_ref, vext_ref, sx_ref)


def _final_body(agg_ref, sxp_ref, out_ref):
    p = agg_ref[0] + agg_ref[1]
    out_ref[...] = p[:, :D] / (p[:, D:D + 1] + 1e-16) + sxp_ref[...]


def _smax_reduce_body(tiles_ref, out_ref):
    g = jnp.max(tiles_ref[...], axis=0)
    out_ref[...] = jnp.where(jnp.isfinite(g), g, 0.0)


def _w_specs():
    shapes = [(D, D), (1, D), (8, D), (1, D), (D, D), (1, D), (D, D), (1, D)]
    return [pl.BlockSpec(s, lambda i: (0, 0)) for s in shapes]


def _tc_prep(h, w):
    return pl.pallas_call(
        _prep_body,
        grid=(GRID,),
        in_specs=[pl.BlockSpec((TB, D), lambda i: (i, 0))] + _w_specs(),
        out_specs=[
            pl.BlockSpec((9, TB), lambda i: (0, i)),
            pl.BlockSpec((TB, VW), lambda i: (i, 0)),
            pl.BlockSpec((TB, D), lambda i: (i, 0)),
        ],
        out_shape=[
            jax.ShapeDtypeStruct((9, NP), _f32),
            jax.ShapeDtypeStruct((N, VW), _f32),
            jax.ShapeDtypeStruct((N, D), _f32),
        ],
    )(h, *w)


def _tc_merge_prep(agg, sxp, w):
    return pl.pallas_call(
        _merge_prep_body,
        grid=(GRID,),
        in_specs=[pl.BlockSpec((NC, TB, VW), lambda i: (0, i, 0)),
                  pl.BlockSpec((TB, D), lambda i: (i, 0))] + _w_specs(),
        out_specs=[
            pl.BlockSpec((9, TB), lambda i: (0, i)),
            pl.BlockSpec((TB, VW), lambda i: (i, 0)),
            pl.BlockSpec((TB, D), lambda i: (i, 0)),
        ],
        out_shape=[
            jax.ShapeDtypeStruct((9, NP), _f32),
            jax.ShapeDtypeStruct((N, VW), _f32),
            jax.ShapeDtypeStruct((N, D), _f32),
        ],
    )(agg, sxp, *w)


def _tc_final(agg, sxp):
    return pl.pallas_call(
        _final_body,
        grid=(GRID,),
        in_specs=[pl.BlockSpec((NC, TB, VW), lambda i: (0, i, 0)),
                  pl.BlockSpec((TB, D), lambda i: (i, 0))],
        out_specs=pl.BlockSpec((TB, D), lambda i: (i, 0)),
        out_shape=jax.ShapeDtypeStruct((N, D), _f32),
    )(agg, sxp)


def _tc_smax_reduce(tiles):
    return pl.pallas_call(
        _smax_reduce_body,
        grid=(1,),
        in_specs=[pl.BlockSpec((NW, N), lambda i: (0, 0))],
        out_specs=pl.BlockSpec((N,), lambda i: (0,)),
        out_shape=jax.ShapeDtypeStruct((N,), _f32),
    )(tiles)


def _sc_pass1_body(qkT, eaT, dst, s_out, smax_tiles,
                   planes_v, ea_v, dst_v, s_v, smax_v):
    c = lax.axis_index("c")
    t = lax.axis_index("s")
    wid = c * NS + t
    base_e = c * EPC + t * EPT

    for d in range(9):
        pltpu.sync_copy(qkT.at[pl.ds(d * NP, N)], planes_v.at[d])

    def _init(i, carry):
        smax_v[pl.ds(i * 16, 16)] = jnp.full((16,), -jnp.inf, _f32)
        return carry
    lax.fori_loop(0, N // 16, _init, 0)

    for blk in range(EPT // BLK1):
        b0 = base_e + blk * BLK1
        gb = c * (EPC // BLK1) + t * (EPT // BLK1) + blk
        pltpu.sync_copy(dst.at[pl.ds(b0, BLK1)], dst_v)
        pltpu.sync_copy(eaT.at[pl.ds(gb * 8 * BLK1, 8 * BLK1)], ea_v)

        def _grp(j, carry):
            dstv = dst_v[pl.ds(j * 16, 16)]
            sacc = plsc.load_gather(planes_v, [jnp.full((16,), 8, _i32), dstv])
            for d in range(8):
                qd = plsc.load_gather(
                    planes_v, [jnp.full((16,), d, _i32), dstv])
                sacc = sacc + qd * ea_v[pl.ds(d * BLK1 + j * 16, 16)]
            s_v[pl.ds(j * 16, 16)] = sacc
            cur = plsc.load_gather(smax_v, [dstv])
            pend = sacc > cur

            def _cond(p):
                return jnp.any(p)

            def _body(p):
                plsc.store_scatter(smax_v, [dstv], sacc, mask=p)
                cur2 = plsc.load_gather(smax_v, [dstv])
                return p & (sacc > cur2)

            lax.while_loop(_cond, _body, pend)
            return carry
        lax.fori_loop(0, BLK1 // 16, _grp, 0)
        pltpu.sync_copy(s_v, s_out.at[pl.ds(b0, BLK1)])

    pltpu.sync_copy(smax_v, smax_tiles.at[pl.ds(wid * N, N)])


def _sc_pass1(qkT, eaT, dst):
    mesh = plsc.VectorSubcoreMesh(core_axis_name="c", subcore_axis_name="s")
    f = pl.kernel(
        _sc_pass1_body,
        out_type=[
            jax.ShapeDtypeStruct((E,), _f32),
            jax.ShapeDtypeStruct((NW * N,), _f32),
        ],
        mesh=mesh,
        scratch_types=[
            pltpu.VMEM((9, N), _f32),
            pltpu.VMEM((8 * BLK1,), _f32),
            pltpu.VMEM((BLK1,), _i32),
            pltpu.VMEM((BLK1,), _f32),
            pltpu.VMEM((N,), _f32),
        ],
        compiler_params=pltpu.CompilerParams(use_tc_tiling_on_sc=False, needs_layout_passes=False),
    )
    return f(qkT, eaT, dst)


NCH = EPT // C2


def _sc_ex_body(s_all, dst, gsmax, ex_out, smax_v, dst_v, s_v):
    c = lax.axis_index("c")
    t = lax.axis_index("s")
    base_e = c * EPC + t * EPT
    pltpu.sync_copy(gsmax, smax_v)
    for blk in range(EPT // BLK1):
        b0 = base_e + blk * BLK1
        pltpu.sync_copy(dst.at[pl.ds(b0, BLK1)], dst_v)
        pltpu.sync_copy(s_all.at[pl.ds(b0, BLK1)], s_v)

        def _grp(j, carry):
            sl = pl.ds(j * 16, 16)
            dstv = dst_v[sl]
            sm = plsc.load_gather(smax_v, [dstv])
            s_v[sl] = jnp.exp(s_v[sl] - sm)
            return carry
        lax.fori_loop(0, BLK1 // 16, _grp, 0)
        pltpu.sync_copy(s_v, ex_out.at[pl.ds(b0, BLK1)])


def _sc_ex(s_all, dst, gsmax):
    mesh = plsc.VectorSubcoreMesh(core_axis_name="c", subcore_axis_name="s")
    f = pl.kernel(
        _sc_ex_body,
        out_type=jax.ShapeDtypeStruct((E,), _f32),
        mesh=mesh,
        scratch_types=[
            pltpu.VMEM((N,), _f32),
            pltpu.VMEM((BLK1,), _i32),
            pltpu.VMEM((BLK1,), _f32),
        ],
        compiler_params=pltpu.CompilerParams(use_tc_tiling_on_sc=False, needs_layout_passes=False),
    )
    return f(s_all, dst, gsmax)


def _sc_pass2_body(src, dst, ex_all, vext, agg,
                   rows0, rows1, rows2,
                   si0, si1, si2, di0, di1, di2, ev0, ev1, ev2, zbuf,
                   ia, ib, ic, ga, gb, gc, sa, sb, sc_, agg_s):
    c = lax.axis_index("c")
    t = lax.axis_index("s")
    base_e = c * EPC + t * EPT

    for r in range(25):
        for g in range(VW // 16):
            zbuf[r, pl.ds(g * 16, 16)] = jnp.zeros((16,), _f32)

    def _z(k, carry):
        pltpu.sync_copy(zbuf, agg_s.at[pl.ds(t * NPT + k * 25, 25)])
        return carry
    lax.fori_loop(0, NPT // 25, _z, 0)
    plsc.subcore_barrier()

    lane0 = lax.broadcasted_iota(_i32, (16,), 0) == 0

    def _idx(ci, si, di, ev, sem):
        cb = base_e + ci * C2
        pltpu.async_copy(src.at[pl.ds(cb, C2)], si, sem)
        pltpu.async_copy(dst.at[pl.ds(cb, C2)], di, sem)
        pltpu.async_copy(ex_all.at[pl.ds(cb, C2)], ev, sem)

    def _idx_wait(ci, si, di, ev, sem):
        cb = base_e + ci * C2
        pltpu.make_async_copy(src.at[pl.ds(cb, C2)], si, sem).wait()
        pltpu.make_async_copy(dst.at[pl.ds(cb, C2)], di, sem).wait()
        pltpu.make_async_copy(ex_all.at[pl.ds(cb, C2)], ev, sem).wait()

    def _compute(rows, ev):
        for e in range(C2):
            exb = plsc.load_gather(ev, [jnp.full((16,), e, _i32)])
            for g in range(D // 16):
                rows[e, pl.ds(g * 16, 16)] = rows[e, pl.ds(g * 16, 16)] * exb
            rows[e, pl.ds(D, 16)] = jnp.where(lane0, exb, 0.0)

    def _gather(si, rows, sem):
        pltpu.async_copy(vext.at[si], rows, sem)

    def _gather_wait(si, rows, sem):
        pltpu.make_async_copy(vext.at[si], rows, sem).wait()

    def _scat(rows, di, sem):
        pltpu.async_copy(rows, agg_s.at[di], sem, add=True)

    def _scat_wait(rows, di, sem):
        pltpu.make_async_copy(rows, agg_s.at[di], sem).wait()

    _idx(0, si0, di0, ev0, ia)
    _idx(1, si1, di1, ev1, ib)
    _idx(2, si2, di2, ev2, ic)
    _idx_wait(0, si0, di0, ev0, ia)
    _gather(si0, rows0, ga)
    _idx_wait(1, si1, di1, ev1, ib)
    _gather(si1, rows1, gb)

    def _body(i, carry):
        j = 3 * i
        _gather_wait(si0, rows0, ga)
        _compute(rows0, ev0)
        _scat(rows0, di0, sa)
        _idx_wait(j + 2, si2, di2, ev2, ic)
        _gather(si2, rows2, gc)
        _gather_wait(si1, rows1, gb)
        _compute(rows1, ev1)
        _scat(rows1, di1, sb)
        _scat_wait(rows0, di0, sa)
        _idx(j + 3, si0, di0, ev0, ia)
        _gather_wait(si2, rows2, gc)
        _compute(rows2, ev2)
        _scat(rows2, di2, sc_)
        _scat_wait(rows1, di1, sb)
        _idx(j + 4, si1, di1, ev1, ib)
        _idx_wait(j + 3, si0, di0, ev0, ia)
        _gather(si0, rows0, ga)
        _scat_wait(rows2, di2, sc_)

        @pl.when(i < NCH // 3 - 1)
        def _():
            _idx(j + 5, si2, di2, ev2, ic)
        _idx_wait(j + 4, si1, di1, ev1, ib)
        _gather(si1, rows1, gb)
        return carry
    lax.fori_loop(0, NCH // 3, _body, 0)

    _gather_wait(si0, rows0, ga)
    _compute(rows0, ev0)
    _scat(rows0, di0, sa)
    _gather_wait(si1, rows1, gb)
    _compute(rows1, ev1)
    _scat(rows1, di1, sb)
    _scat_wait(rows0, di0, sa)
    _scat_wait(rows1, di1, sb)
    plsc.subcore_barrier()

    pltpu.sync_copy(agg_s.at[pl.ds(t * NPT, NPT)],
                    agg.at[c, pl.ds(t * NPT, NPT)])


def _sc_pass2(src, dst, ex_all, vext):
    mesh = plsc.VectorSubcoreMesh(core_axis_name="c", subcore_axis_name="s")
    f = pl.kernel(
        _sc_pass2_body,
        out_type=jax.ShapeDtypeStruct((NC, N, VW), _f32),
        mesh=mesh,
        scratch_types=(
            [pltpu.VMEM((C2, VW), _f32)] * 3
            + [pltpu.VMEM((C2,), _i32)] * 6
            + [pltpu.VMEM((C2,), _f32)] * 3
            + [pltpu.VMEM((25, VW), _f32)]
            + [pltpu.SemaphoreType.DMA] * 9
            + [pltpu.VMEM_SHARED((N, VW), _f32)]
        ),
        compiler_params=pltpu.CompilerParams(use_tc_tiling_on_sc=False, needs_layout_passes=False),
    )
    return f(src, dst, ex_all, vext)


def kernel(x, edge_index, edge_attr, params):
    src = edge_index[0]
    dst = edge_index[1]
    eaT = (edge_attr.T.reshape(8, E // BLK1, BLK1)
           .transpose(1, 0, 2).reshape(-1))

    def weights(p):
        return (p["Wq"], p["bq"].reshape(1, D), p["Wk"], p["bk"].reshape(1, D),
                p["Wv"], p["bv"].reshape(1, D), p["Ws"], p["bs"].reshape(1, D))

    agg = None
    sx = None
    for li in range(3):
        w = weights(params[li])
        if li == 0:
            qkT, vext, sx = _tc_prep(x, w)
        else:
            qkT, vext, sx = _tc_merge_prep(agg, sx, w)
        s_all, smax_tiles = _sc_pass1(qkT.reshape(-1), eaT, dst)
        gsmax = _tc_smax_reduce(smax_tiles.reshape(NW, N))
        ex_all = _sc_ex(s_all, dst, gsmax)
        agg = _sc_pass2(src, dst, ex_all, vext)
    return _tc_final(agg, sx)
